# baseline (speedup 1.0000x reference)
import jax
import jax.numpy as jnp
from jax import lax
from jax.experimental import pallas as pl
from jax.experimental.pallas import tpu as pltpu
from jax.experimental.pallas import tpu_sc as plsc

B, H, W = 4, 1024, 1920
N = H * W
NC, NS, L = 2, 16, 16
ROWS_PER_TILE = H // NS
CHUNK = ROWS_PER_TILE * W
WIN = 384
SUBW = W // WIN
NWIN = ROWS_PER_TILE * SUBW
NJ = WIN // 128
GROUPS = WIN // L
ZCHUNK = CHUNK // 8
BATCH_PER_CORE = B // NC


def _splat_body(px_hbm, fx_hbm, fy_hbm, w_hbm, zeros_hbm,
                cnt_hbm, o0_hbm, o1_hbm,
                acc, pxb, fb, wb, valb, idxb, drainb, sem_in, sem_sc):
    c = lax.axis_index("c")
    t = lax.axis_index("s")
    toff = t * CHUNK

    def fire_inputs(a_static, b, w, p):
        base = b * N + (t * ROWS_PER_TILE + w // SUBW) * W + (w % SUBW) * WIN
        pltpu.async_copy(px_hbm.at[pl.ds(base, WIN)], pxb.at[p], sem_in)
        pltpu.async_copy(w_hbm.at[pl.ds(base, WIN)], wb.at[p], sem_in)
        if a_static == 1:
            pltpu.async_copy(fx_hbm.at[pl.ds(base, WIN)], fb.at[p], sem_in)
        elif a_static == 2:
            pltpu.async_copy(fy_hbm.at[pl.ds(base, WIN)], fb.at[p], sem_in)

    def wait_inputs(a_static, p):
        pltpu.make_async_copy(px_hbm.at[pl.ds(0, WIN)], pxb.at[p],
                              sem_in).wait()
        pltpu.make_async_copy(w_hbm.at[pl.ds(0, WIN)], wb.at[p],
                              sem_in).wait()
        if a_static != 0:
            pltpu.make_async_copy(w_hbm.at[pl.ds(0, WIN)], fb.at[p],
                                  sem_in).wait()

    def drain_scatters():
        pltpu.make_async_copy(
            zeros_hbm.at[pl.ds(0, 4 * WIN)], drainb, sem_sc).wait()

    def do_round(a_static, b, out_ref):
        def zero_body(z, carry):
            pltpu.sync_copy(zeros_hbm, acc.at[pl.ds(toff + z * ZCHUNK, ZCHUNK)])
            return carry

        lax.fori_loop(0, CHUNK // ZCHUNK, zero_body, 0)
        plsc.subcore_barrier()

        fire_inputs(a_static, b, 0, 0)

        def win_loop(w, carry):
            p = lax.rem(w, 2)
            wait_inputs(a_static, p)

            @pl.when(w + 1 < NWIN)
            def _():
                fire_inputs(a_static, b, w + 1, 1 - p)

            @pl.when(w >= 2)
            def _():
                drain_scatters()

            pxw = pxb.at[p]
            fw = fb.at[p]
            ww = wb.at[p]
            valw = valb.at[p]
            idxw = idxb.at[p]

            def group(g, gcarry):
                s = g * L
                pxv = pxw[pl.ds(s, L)]
                wv = ww[pl.ds(s, L)]
                i0 = lax.shift_right_logical(pxv, 3)
                dx = lax.shift_right_logical(pxv & 2, 1)
                dwv = (pxv & 4) * (W // 4)
                i2 = i0 + dwv
                wd = jnp.where((pxv & 1) > 0, wv, 0.0)
                if a_static == 0:
                    v = wd
                else:
                    v = fw[pl.ds(s, L)] * (-wd)
                valw[pl.ds(s, L)] = v
                j = g // 8
                col = (g % 8) * L
                idxw[j, pl.ds(col, L)] = i0
                idxw[NJ + j, pl.ds(col, L)] = i0 + dx
                idxw[2 * NJ + j, pl.ds(col, L)] = i2
                idxw[3 * NJ + j, pl.ds(col, L)] = i2 + dx
                return gcarry

            lax.fori_loop(0, GROUPS, group, 0)

            def scat(j, scarry):
                src = valw.at[pl.ds(j * 128, 128)]
                pltpu.async_copy(src, acc.at[idxw.at[j]], sem_sc, add=True)
                pltpu.async_copy(src, acc.at[idxw.at[NJ + j]], sem_sc,
                                 add=True)
                pltpu.async_copy(src, acc.at[idxw.at[2 * NJ + j]], sem_sc,
                                 add=True)
                pltpu.async_copy(src, acc.at[idxw.at[3 * NJ + j]], sem_sc,
                                 add=True)
                return scarry

            lax.fori_loop(0, NJ, scat, 0)
            return carry

        lax.fori_loop(0, NWIN, win_loop, 0)
        drain_scatters()
        drain_scatters()
        plsc.subcore_barrier()

        def copyout(z, carry):
            pltpu.sync_copy(
                acc.at[pl.ds(toff + z * ZCHUNK, ZCHUNK)],
                out_ref.at[pl.ds(b * N + toff + z * ZCHUNK, ZCHUNK)])
            return carry

        lax.fori_loop(0, CHUNK // ZCHUNK, copyout, 0)

    for a_static, out_ref in ((0, cnt_hbm), (1, o0_hbm), (2, o1_hbm)):
        def rounds(bi, carry, _a=a_static, _o=out_ref):
            do_round(_a, c * BATCH_PER_CORE + bi, _o)
            return carry

        lax.fori_loop(0, BATCH_PER_CORE, rounds, 0)


HC = 128


def _rec_body(fx_ref, fy_ref, px_ref):
    h = pl.program_id(1)
    fx = fx_ref[0]
    fy = fy_ref[0]
    xs = lax.broadcasted_iota(jnp.int32, (HC, W), 1).astype(jnp.float32)
    ys = (lax.broadcasted_iota(jnp.int32, (HC, W), 0) + h * HC
          ).astype(jnp.float32)
    x2 = xs + fx
    y2 = ys + fy
    valid = ((x2 >= 0.0) & (y2 >= 0.0)
             & (x2 <= float(W - 1)) & (y2 <= float(H - 1)))
    x2c = jnp.clip(x2, 0.0, float(W - 1))
    y2c = jnp.clip(y2, 0.0, float(H - 1))
    ixL = x2c.astype(jnp.int32)
    iyT = y2c.astype(jnp.int32)
    dx = jnp.minimum(ixL + 1, W - 1) - ixL
    dy = jnp.minimum(iyT + 1, H - 1) - iyT
    i0 = iyT * W + ixL
    px_ref[0] = i0 * 8 + dy * 4 + dx * 2 + valid.astype(jnp.int32)


def _records(fx, fy):
    spec = pl.BlockSpec((1, HC, W), lambda b, h: (b, h, 0))
    return pl.pallas_call(
        _rec_body,
        grid=(B, H // HC),
        in_specs=[spec, spec],
        out_specs=spec,
        out_shape=jax.ShapeDtypeStruct((B, H, W), jnp.int32),
    )(fx, fy)


def _norm_body(cnt_ref, o0_ref, o1_ref, out_ref):
    cv = cnt_ref[0]
    m = cv > 0.0
    d = jnp.where(m, cv, 1.0)
    o0 = o0_ref[0]
    o1 = o1_ref[0]
    out_ref[0, 0] = jnp.where(m, o0 / d, o0)
    out_ref[0, 1] = jnp.where(m, o1 / d, o1)


def _normalize(cnt, o0, o1):
    spec3 = pl.BlockSpec((1, HC, W), lambda b, h: (b, h, 0))
    return pl.pallas_call(
        _norm_body,
        grid=(B, H // HC),
        in_specs=[spec3, spec3, spec3],
        out_specs=pl.BlockSpec((1, 2, HC, W), lambda b, h: (b, 0, h, 0)),
        out_shape=jax.ShapeDtypeStruct((B, 2, H, W), jnp.float32),
    )(cnt, o0, o1)


@jax.jit
def kernel(input1, input2):
    fx = input1[:, 0]
    fy = input1[:, 1]
    px = _records(fx, fy)
    zeros = jnp.zeros((ZCHUNK,), jnp.float32)
    mesh = plsc.VectorSubcoreMesh(core_axis_name="c", subcore_axis_name="s")
    cnt, o0, o1 = pl.kernel(
        _splat_body,
        out_type=[jax.ShapeDtypeStruct((B * N,), jnp.float32)] * 3,
        mesh=mesh,
        scratch_types=[
            pltpu.VMEM_SHARED((N,), jnp.float32),
            pltpu.VMEM((2, WIN), jnp.int32),
            pltpu.VMEM((2, WIN), jnp.float32),
            pltpu.VMEM((2, WIN), jnp.float32),
            pltpu.VMEM((2, WIN), jnp.float32),
            pltpu.VMEM((2, 4 * NJ, 128), jnp.int32),
            pltpu.VMEM((4 * WIN,), jnp.float32),
            pltpu.SemaphoreType.DMA,
            pltpu.SemaphoreType.DMA,
        ],
    )(px.reshape(-1), fx.reshape(-1), fy.reshape(-1),
      input2[:, 0].reshape(-1), zeros)
    return _normalize(cnt.reshape(B, H, W), o0.reshape(B, H, W),
                      o1.reshape(B, H, W))


# speedup vs baseline: 1.1874x; 1.1874x over previous
import jax
import jax.numpy as jnp
from jax import lax
from jax.experimental import pallas as pl
from jax.experimental.pallas import tpu as pltpu
from jax.experimental.pallas import tpu_sc as plsc

B, H, W = 4, 1024, 1920
N = H * W
NC, NS, L = 2, 16, 16
ROWS_PER_TILE = H // NS
CHUNK = ROWS_PER_TILE * W
WIN = 640
SUBW = W // WIN
NWIN = ROWS_PER_TILE * SUBW
NJ = WIN // 128
GROUPS = WIN // L
ZCHUNK = CHUNK // 8
NB = 2


def _splat_body(px_hbm, v0_hbm, v1_hbm, v2_hbm, zeros_hbm,
                cnt_hbm, o0_hbm, o1_hbm,
                acc, pxb, valb, idxb, sem_in, sem_sc):
    c = lax.axis_index("c")
    t = lax.axis_index("s")
    toff = t * CHUNK

    def fire_inputs(val_hbm, b, w, p):
        base = b * N + (t * ROWS_PER_TILE + w // SUBW) * W + (w % SUBW) * WIN
        pltpu.async_copy(px_hbm.at[pl.ds(base, WIN)], pxb.at[p], sem_in)
        pltpu.async_copy(val_hbm.at[pl.ds(base, WIN)], valb.at[p], sem_in)

    def wait_inputs(p):
        pltpu.make_async_copy(px_hbm.at[pl.ds(0, WIN)], pxb.at[p],
                              sem_in).wait()
        pltpu.make_async_copy(v0_hbm.at[pl.ds(0, WIN)], valb.at[p],
                              sem_in).wait()

    def drain_scatters():
        for _ in range(4):
            pltpu.make_async_copy(
                zeros_hbm.at[pl.ds(0, WIN)], valb.at[0], sem_sc).wait()

    def do_round(val_hbm, b, out_ref):
        def zero_body(z, carry):
            pltpu.sync_copy(zeros_hbm, acc.at[pl.ds(toff + z * ZCHUNK, ZCHUNK)])
            return carry

        lax.fori_loop(0, CHUNK // ZCHUNK, zero_body, 0)
        plsc.subcore_barrier()

        fire_inputs(val_hbm, b, 0, 0)

        def win_loop(w, carry):
            p = lax.rem(w, 2)
            wait_inputs(p)

            @pl.when(w + 1 < NWIN)
            def _():
                fire_inputs(val_hbm, b, w + 1, 1 - p)

            @pl.when(w >= 2)
            def _():
                drain_scatters()

            pxw = pxb.at[p]
            valw = valb.at[p]
            po = p * 4 * NJ

            def group(g, gcarry):
                s = g * L
                pxv = pxw[pl.ds(s, L)]
                i0 = lax.shift_right_logical(pxv, 2)
                dx = pxv & 1
                dwv = (pxv & 2) * (W // 2)
                i2 = i0 + dwv
                j = po + g // 8
                col = (g % 8) * L
                idxb[j, pl.ds(col, L)] = i0
                idxb[NJ + j, pl.ds(col, L)] = i0 + dx
                idxb[2 * NJ + j, pl.ds(col, L)] = i2
                idxb[3 * NJ + j, pl.ds(col, L)] = i2 + dx
                return gcarry

            lax.fori_loop(0, GROUPS, group, 0)

            def scat(j, scarry):
                src = valw.at[pl.ds(j * 128, 128)]
                jj = po + j
                pltpu.async_copy(src, acc.at[idxb.at[jj]], sem_sc, add=True)
                pltpu.async_copy(src, acc.at[idxb.at[NJ + jj]], sem_sc,
                                 add=True)
                pltpu.async_copy(src, acc.at[idxb.at[2 * NJ + jj]], sem_sc,
                                 add=True)
                pltpu.async_copy(src, acc.at[idxb.at[3 * NJ + jj]], sem_sc,
                                 add=True)
                return scarry

            lax.fori_loop(0, NJ, scat, 0)
            return carry

        lax.fori_loop(0, NWIN, win_loop, 0)
        drain_scatters()
        drain_scatters()
        plsc.subcore_barrier()

        def copyout(z, carry):
            pltpu.sync_copy(
                acc.at[pl.ds(toff + z * ZCHUNK, ZCHUNK)],
                out_ref.at[pl.ds(b * N + toff + z * ZCHUNK, ZCHUNK)])
            return carry

        lax.fori_loop(0, CHUNK // ZCHUNK, copyout, 0)

    for val_hbm, out_ref in ((v0_hbm, cnt_hbm), (v1_hbm, o0_hbm),
                             (v2_hbm, o1_hbm)):
        do_round(val_hbm, c, out_ref)


HC = 128


def _rec_body(fx_ref, fy_ref, w_ref, px_ref, v0_ref, v1_ref, v2_ref):
    h = pl.program_id(1)
    fx = fx_ref[0]
    fy = fy_ref[0]
    wv = w_ref[0]
    xs = lax.broadcasted_iota(jnp.int32, (HC, W), 1).astype(jnp.float32)
    ys = (lax.broadcasted_iota(jnp.int32, (HC, W), 0) + h * HC
          ).astype(jnp.float32)
    x2 = xs + fx
    y2 = ys + fy
    valid = ((x2 >= 0.0) & (y2 >= 0.0)
             & (x2 <= float(W - 1)) & (y2 <= float(H - 1)))
    x2c = jnp.clip(x2, 0.0, float(W - 1))
    y2c = jnp.clip(y2, 0.0, float(H - 1))
    ixL = x2c.astype(jnp.int32)
    iyT = y2c.astype(jnp.int32)
    dx = jnp.minimum(ixL + 1, W - 1) - ixL
    dy = jnp.minimum(iyT + 1, H - 1) - iyT
    i0 = iyT * W + ixL
    px_ref[0] = i0 * 4 + dy * 2 + dx
    wd = jnp.where(valid, wv, 0.0)
    v0_ref[0] = wd
    v1_ref[0] = -fx * wd
    v2_ref[0] = -fy * wd


def _records(fx, fy, wv):
    spec = pl.BlockSpec((1, HC, W), lambda b, h: (b, h, 0))
    return pl.pallas_call(
        _rec_body,
        grid=(NB, H // HC),
        in_specs=[spec, spec, spec],
        out_specs=[spec, spec, spec, spec],
        out_shape=[jax.ShapeDtypeStruct((NB, H, W), jnp.int32)]
        + [jax.ShapeDtypeStruct((NB, H, W), jnp.float32)] * 3,
    )(fx, fy, wv)


def _norm_body(cnt_ref, o0_ref, o1_ref, out_ref):
    cv = cnt_ref[0]
    m = cv > 0.0
    d = jnp.where(m, cv, 1.0)
    o0 = o0_ref[0]
    o1 = o1_ref[0]
    out_ref[0, 0] = jnp.where(m, o0 / d, o0)
    out_ref[0, 1] = jnp.where(m, o1 / d, o1)


def _normalize(cnt, o0, o1):
    spec3 = pl.BlockSpec((1, HC, W), lambda b, h: (b, h, 0))
    return pl.pallas_call(
        _norm_body,
        grid=(NB, H // HC),
        in_specs=[spec3, spec3, spec3],
        out_specs=pl.BlockSpec((1, 2, HC, W), lambda b, h: (b, 0, h, 0)),
        out_shape=jax.ShapeDtypeStruct((NB, 2, H, W), jnp.float32),
    )(cnt, o0, o1)


def _splat(px, v0, v1, v2, zeros):
    mesh = plsc.VectorSubcoreMesh(core_axis_name="c", subcore_axis_name="s")
    return pl.kernel(
        _splat_body,
        out_type=[jax.ShapeDtypeStruct((NB * N,), jnp.float32)] * 3,
        mesh=mesh,
        scratch_types=[
            pltpu.VMEM_SHARED((N,), jnp.float32),
            pltpu.VMEM((2, WIN), jnp.int32),
            pltpu.VMEM((2, WIN), jnp.float32),
            pltpu.VMEM((8 * NJ, 128), jnp.int32),
            pltpu.SemaphoreType.DMA,
            pltpu.SemaphoreType.DMA,
        ],
    )(px.reshape(-1), v0.reshape(-1), v1.reshape(-1), v2.reshape(-1), zeros)


def _half(input1, input2, zeros, lo):
    px, v0, v1, v2 = _records(input1[lo:lo + NB, 0], input1[lo:lo + NB, 1],
                              input2[lo:lo + NB, 0])
    cnt, o0, o1 = _splat(px, v0, v1, v2, zeros)
    return _normalize(cnt.reshape(NB, H, W), o0.reshape(NB, H, W),
                      o1.reshape(NB, H, W))


@jax.jit
def kernel(input1, input2):
    zeros = jnp.zeros((ZCHUNK,), jnp.float32)
    out01 = _half(input1, input2, zeros, 0)
    out23 = _half(input1, input2, zeros, 2)
    return jnp.concatenate([out01, out23], axis=0)
